# Initial kernel scaffold; baseline (speedup 1.0000x reference)
import jax
import jax.numpy as jnp
from jax.experimental import pallas as pl


def kernel(ctx_embeddings, ctx_begin_pos, input_ids, token_table, pos_table):
    raise NotImplementedError("write your pallas kernel here")


# speedup vs baseline: 4.7863x; 4.7863x over previous
import jax
import jax.numpy as jnp
from jax import lax
from jax.experimental import pallas as pl
from jax.experimental.pallas import tpu as pltpu
from jax.experimental.pallas import tpu_sc as plsc

VOCAB = 49408
MAX_POS = 128
D = 768
B = 1024
S = 64
C = 16
TOTAL = S + C

NC = 2
NS = 16
NW = NC * NS
BPW = B // NW
L = 16
DCH = D // L


def _body(cbp_hbm, ids_hbm, ctx_hbm, tok_hbm, pos_hbm, out_hbm,
          cbp_v, ids_v, posidx_v, tokdst_v, ctxdst_v,
          pos_vmem, tbuf, cbuf,
          sem_t, sem_c, sem_o1, sem_o2):
  wid = lax.axis_index("s") * NC + lax.axis_index("c")
  base = wid * BPW

  pltpu.sync_copy(cbp_hbm.at[pl.ds(base, BPW)], cbp_v)
  pltpu.sync_copy(ids_hbm.at[pl.ds(base, BPW)], ids_v)
  pltpu.sync_copy(pos_hbm.at[pl.ds(0, TOTAL)], pos_vmem)

  iota = lax.iota(jnp.int32, L)

  def sample(i, carry):
    b = base + i
    cbp = plsc.load_gather(cbp_v, [jnp.full((L,), i, jnp.int32)])
    b_total = jnp.full((L,), b * TOTAL, jnp.int32)

    for k in range(S // L):
      t = iota + (k * L)
      pi = jnp.where(t >= cbp, t + C, t)
      posidx_v[pl.ds(k * L, L)] = pi
      tokdst_v[pl.ds(k * L, L)] = pi + b_total
    ctxdst_v[...] = cbp + iota + b_total

    c1 = pltpu.async_copy(tok_hbm.at[ids_v.at[i]], tbuf, sem_t)
    c2 = pltpu.async_copy(ctx_hbm.at[pl.ds(b * C, C)], cbuf, sem_c)
    c1.wait()
    c2.wait()

    def add_tok(r, carry2):
      rs = jnp.full((L,), r, jnp.int32)
      prow = plsc.load_gather(posidx_v, [rs])
      for k in range(DCH):
        col = iota + (k * L)
        pv = plsc.load_gather(pos_vmem, [prow, col])
        plsc.addupdate_scatter(tbuf, [rs, col], pv)
      return carry2
    lax.fori_loop(0, S, add_tok, 0)

    def add_ctx(r, carry2):
      rs = jnp.full((L,), r, jnp.int32)
      prow = cbp + rs
      for k in range(DCH):
        col = iota + (k * L)
        pv = plsc.load_gather(pos_vmem, [prow, col])
        plsc.addupdate_scatter(cbuf, [rs, col], pv)
      return carry2
    lax.fori_loop(0, C, add_ctx, 0)

    s1 = pltpu.async_copy(tbuf, out_hbm.at[tokdst_v], sem_o1)
    s2 = pltpu.async_copy(cbuf, out_hbm.at[ctxdst_v], sem_o2)
    s1.wait()
    s2.wait()
    return carry

  lax.fori_loop(0, BPW, sample, 0)


@jax.jit
def _sc_embed(cbp, ids, ctx2, token_table, pos_table):
  mesh = plsc.VectorSubcoreMesh(
      core_axis_name="c", subcore_axis_name="s", num_cores=NC, num_subcores=NS)
  f = pl.kernel(
      _body,
      out_type=jax.ShapeDtypeStruct((B * TOTAL, D), jnp.float32),
      mesh=mesh,
      compiler_params=pltpu.CompilerParams(needs_layout_passes=False),
      scratch_types=[
          pltpu.VMEM((BPW,), jnp.int32),
          pltpu.VMEM((BPW, S), jnp.int32),
          pltpu.VMEM((S,), jnp.int32),
          pltpu.VMEM((S,), jnp.int32),
          pltpu.VMEM((C,), jnp.int32),
          pltpu.VMEM((TOTAL, D), jnp.float32),
          pltpu.VMEM((S, D), jnp.float32),
          pltpu.VMEM((C, D), jnp.float32),
          pltpu.SemaphoreType.DMA,
          pltpu.SemaphoreType.DMA,
          pltpu.SemaphoreType.DMA,
          pltpu.SemaphoreType.DMA,
      ],
  )
  return f(cbp, ids, ctx2, token_table, pos_table)


def kernel(ctx_embeddings, ctx_begin_pos, input_ids, token_table, pos_table):
  ctx2 = ctx_embeddings.reshape(B * C, D)
  ids = input_ids.astype(jnp.int32)
  cbp = ctx_begin_pos.astype(jnp.int32)
  out = _sc_embed(cbp, ids, ctx2, token_table, pos_table)
  return out.reshape(B, TOTAL, D)
